# baseline (speedup 1.0000x reference)
import functools

import numpy as np

import jax
import jax.numpy as jnp
from jax.experimental import pallas as pl
from jax.experimental.pallas import tpu as pltpu


def _fused_body(rr, ss, oo, p1, dz, l1, f2, l2, da,
                a_hbm, x_ref, a16_ref, w_ref, o_ref,
                abuf, xy_sc, z_sc, stage, sem,
                *, bm, cwb, sb, sw, f, nsteps):
    t = pl.program_id(0)
    r = rr[t]
    s = pl.multiple_of(ss[t], 128)
    rb = pl.multiple_of(r * bm, 8)

    def _copy(step, slot):
        return pltpu.make_async_copy(
            a_hbm.at[pl.ds(pl.multiple_of(rr[step] * bm, 8), bm),
                     pl.ds(pl.multiple_of(ss[step], 128), cwb)],
            abuf.at[slot], sem.at[slot])

    @pl.when(t == 0)
    def _():
        z_sc[...] = jnp.zeros(z_sc.shape, z_sc.dtype)
        xy_sc[:, f:] = jnp.zeros((xy_sc.shape[0], f), xy_sc.dtype)
        xy_sc[:, :f] = x_ref[...]
        _copy(0, 0).start()
        _copy(1, 1).start()
        _copy(2, 2).start()

    @pl.when(t + 3 < nsteps)
    def _():
        _copy(t + 3, jax.lax.rem(t + 3, 4)).start()

    _copy(t, jax.lax.rem(t, 4)).wait()

    def _work(ab_ref):
        @pl.when(p1[t] == 1)
        def _():
            term = jnp.dot(ab_ref[...], xy_sc[pl.ds(s, cwb), :],
                           preferred_element_type=jnp.float32)

            @pl.when(s == 0)
            def _():
                stage[...] = term[:, :f]

            @pl.when(s != 0)
            def _():
                stage[...] += term[:, :f]

            @pl.when(dz[t] == 1)
            def _():
                z_sc[pl.ds(rb, bm), :] += term[:, f:]

        @pl.when(da[t] == 1)
        def _():
            o_ref[...] += jnp.dot(ab_ref[...], xy_sc[pl.ds(s, cwb), f:],
                                  preferred_element_type=jnp.float32)

    @pl.when(f2[t] == 1)
    def _():
        o_ref[...] = z_sc[pl.ds(rb, bm), :]

    @pl.when(jax.lax.rem(t, 4) == 0)
    def _():
        _work(abuf.at[0])

    @pl.when(jax.lax.rem(t, 4) == 1)
    def _():
        _work(abuf.at[1])

    @pl.when(jax.lax.rem(t, 4) == 2)
    def _():
        _work(abuf.at[2])

    @pl.when(jax.lax.rem(t, 4) == 3)
    def _():
        _work(abuf.at[3])

    @pl.when(l1[t] == 1)
    def _():
        stage[...] += jnp.dot(a16_ref[...], x_ref[pl.ds(sb, sw), :],
                              preferred_element_type=jnp.float32)
        xy_sc[pl.ds(rb, bm), f:] = stage[...]

    @pl.when(l2[t] == 1)
    def _():
        acc = o_ref[...] + jnp.dot(a16_ref[...], xy_sc[pl.ds(sb, sw), f:],
                                   preferred_element_type=jnp.float32)
        o_ref[...] = 0.25 * jnp.dot(acc, w_ref[...],
                                    preferred_element_type=jnp.float32)


def _fused(feature, adj_self, weight, bm, cwb):
    n, f = feature.shape
    e = weight.shape[1]
    nr = n // bm
    sb = (n // cwb) * cwb if n % cwb else n - cwb
    ncb = sb // cwb
    sw = n - sb

    rr, ss, oo, p1, dz, l1, f2, l2, da = ([] for _ in range(9))

    def add(r, s, p1_, dz_, l1_, f2_, l2_, da_=0):
        rr.append(r); ss.append(s); oo.append(r if not p1_ else 0)
        p1.append(p1_); dz.append(dz_)
        l1.append(l1_); f2.append(f2_); l2.append(l2_); da.append(da_)

    for r in range(nr):
        for c in range(ncb):
            add(r, c * cwb, 1, 1 if cwb * (c + 1) <= bm * r else 0,
                1 if c == ncb - 1 else 0, 0, 0)
    for r in range(nr):
        cnt = (bm * r) // cwb
        if cnt >= ncb:
            add(r, (ncb - 1) * cwb, 0, 0, 0, 1, 1, 0)
        else:
            for c in range(cnt, ncb):
                add(r, c * cwb, 0, 0, 0,
                    1 if c == cnt else 0, 1 if c == ncb - 1 else 0, 1)
    nsteps = len(rr)

    arrs = [jnp.asarray(np.asarray(a, dtype=np.int32))
            for a in (rr, ss, oo, p1, dz, l1, f2, l2, da)]
    a16 = jax.lax.slice(adj_self, (0, sb), (n, n))
    body = functools.partial(_fused_body, bm=bm, cwb=cwb, sb=sb, sw=sw,
                             f=f, nsteps=nsteps)
    grid_spec = pltpu.PrefetchScalarGridSpec(
        num_scalar_prefetch=9,
        grid=(nsteps,),
        in_specs=[
            pl.BlockSpec(memory_space=pltpu.MemorySpace.HBM),
            pl.BlockSpec((n, f), lambda t, *_: (0, 0)),
            pl.BlockSpec((bm, sw), lambda t, rr, *_: (rr[t], 0)),
            pl.BlockSpec((f, e), lambda t, *_: (0, 0)),
        ],
        out_specs=pl.BlockSpec((bm, e),
                               lambda t, rr, ss, oo, *_: (oo[t], 0)),
        scratch_shapes=[
            pltpu.VMEM((4, bm, cwb), jnp.float32),
            pltpu.VMEM((n, 2 * f), jnp.float32),
            pltpu.VMEM((n, f), jnp.float32),
            pltpu.VMEM((bm, f), jnp.float32),
            pltpu.SemaphoreType.DMA((4,)),
        ],
    )
    return pl.pallas_call(
        body,
        grid_spec=grid_spec,
        out_shape=jax.ShapeDtypeStruct((n, e), jnp.float32),
    )(*arrs, adj_self, feature, a16, weight)


def _pick_bm(n):
    for bm in (512, 400, 256, 200, 128, 80, 64, 40, 16, 8):
        if n % bm == 0:
            return bm
    return n


def _spmm_kernel(a_ref, x_ref, o_ref):
    o_ref[...] = jnp.dot(a_ref[...], x_ref[...],
                         preferred_element_type=jnp.float32)


def _spmm_proj_kernel(a_ref, y_ref, w_ref, o_ref):
    t = jnp.dot(a_ref[...], y_ref[...], preferred_element_type=jnp.float32)
    o_ref[...] = 0.25 * jnp.dot(t, w_ref[...],
                                preferred_element_type=jnp.float32)


def _two_pass(feature, adj_self, weight):
    n, f = feature.shape
    e = weight.shape[1]
    bm = _pick_bm(n)
    grid = (n // bm,)
    params = pltpu.CompilerParams(dimension_semantics=("parallel",))
    y = pl.pallas_call(
        _spmm_kernel,
        grid=grid,
        in_specs=[
            pl.BlockSpec((bm, n), lambda i: (i, 0)),
            pl.BlockSpec((n, f), lambda i: (0, 0)),
        ],
        out_specs=pl.BlockSpec((bm, f), lambda i: (i, 0)),
        out_shape=jax.ShapeDtypeStruct((n, f), jnp.float32),
        compiler_params=params,
    )(adj_self, feature)
    z = pl.pallas_call(
        _spmm_proj_kernel,
        grid=grid,
        in_specs=[
            pl.BlockSpec((bm, n), lambda i: (i, 0)),
            pl.BlockSpec((n, f), lambda i: (0, 0)),
            pl.BlockSpec((f, e), lambda i: (0, 0)),
        ],
        out_specs=pl.BlockSpec((bm, e), lambda i: (i, 0)),
        out_shape=jax.ShapeDtypeStruct((n, e), jnp.float32),
        compiler_params=params,
    )(adj_self, y, weight)
    return z


def kernel(feature, adj_self, weight):
    n = feature.shape[0]
    if (adj_self.shape == (n, n) and n % 25 == 0 and (n // 25) % 8 == 0
            and n % 128 != 0 and (n % 128) % 8 == 0):
        t128 = (n - n % 128) // 128
        for ncb in (6, 3, 2, 4, 5, 7, 8, 9, 10, 13):
            if t128 % ncb == 0:
                cwb = (t128 // ncb) * 128
                return _fused(feature, adj_self, weight, n // 25, cwb)
    return _two_pass(feature, adj_self, weight)


# speedup vs baseline: 1.0002x; 1.0002x over previous
import functools

import numpy as np

import jax
import jax.numpy as jnp
from jax.experimental import pallas as pl
from jax.experimental.pallas import tpu as pltpu


def _fused_body(rr, ss, oo, p1, dz, l1, f2, l2, da,
                a_hbm, x_ref, a16_ref, w_ref, o_ref,
                abuf, xy_sc, z_sc, stage, sem,
                *, bm, cwb, sb, sw, f, nsteps):
    t = pl.program_id(0)
    r = rr[t]
    s = pl.multiple_of(ss[t], 128)
    rb = pl.multiple_of(r * bm, 8)

    def _copy(step, slot):
        return pltpu.make_async_copy(
            a_hbm.at[pl.ds(pl.multiple_of(rr[step] * bm, 8), bm),
                     pl.ds(pl.multiple_of(ss[step], 128), cwb)],
            abuf.at[slot], sem.at[slot])

    @pl.when(t == 0)
    def _():
        z_sc[...] = jnp.zeros(z_sc.shape, z_sc.dtype)
        xy_sc[:, f:] = jnp.zeros((xy_sc.shape[0], f), xy_sc.dtype)
        xy_sc[:, :f] = x_ref[...]
        _copy(0, 0).start()
        _copy(1, 1).start()
        _copy(2, 2).start()

    @pl.when(t + 3 < nsteps)
    def _():
        _copy(t + 3, jax.lax.rem(t + 3, 4)).start()

    _copy(t, jax.lax.rem(t, 4)).wait()

    def _work(ab_ref):
        @pl.when(p1[t] == 1)
        def _():
            term = jnp.dot(ab_ref[...], xy_sc[pl.ds(s, cwb), :],
                           preferred_element_type=jnp.float32)

            @pl.when(s == 0)
            def _():
                stage[...] = term[:, :f]

            @pl.when(s != 0)
            def _():
                stage[...] += term[:, :f]

            @pl.when(dz[t] == 1)
            def _():
                z_sc[pl.ds(rb, bm), :] += term[:, f:]

        @pl.when(da[t] == 1)
        def _():
            o_ref[...] += jnp.dot(ab_ref[...], xy_sc[pl.ds(s, cwb), f:],
                                  preferred_element_type=jnp.float32)

    @pl.when(f2[t] == 1)
    def _():
        o_ref[...] = z_sc[pl.ds(rb, bm), :]

    @pl.when(jax.lax.rem(t, 4) == 0)
    def _():
        _work(abuf.at[0])

    @pl.when(jax.lax.rem(t, 4) == 1)
    def _():
        _work(abuf.at[1])

    @pl.when(jax.lax.rem(t, 4) == 2)
    def _():
        _work(abuf.at[2])

    @pl.when(jax.lax.rem(t, 4) == 3)
    def _():
        _work(abuf.at[3])

    @pl.when(l1[t] == 1)
    def _():
        stage[...] += jnp.dot(a16_ref[...], x_ref[pl.ds(sb, sw), :],
                              preferred_element_type=jnp.float32)
        xy_sc[pl.ds(rb, bm), f:] = stage[...]

    @pl.when(l2[t] == 1)
    def _():
        acc = o_ref[...] + jnp.dot(a16_ref[...], xy_sc[pl.ds(sb, sw), f:],
                                   preferred_element_type=jnp.float32)
        o_ref[...] = 0.25 * jnp.dot(acc, w_ref[...],
                                    preferred_element_type=jnp.float32)


def _fused(feature, adj_self, weight, bm, cwb):
    n, f = feature.shape
    e = weight.shape[1]
    nr = n // bm
    sb = (n // cwb) * cwb if n % cwb else n - cwb
    ncb = sb // cwb
    sw = n - sb

    rr, ss, oo, p1, dz, l1, f2, l2, da = ([] for _ in range(9))

    def add(r, s, p1_, dz_, l1_, f2_, l2_, da_=0):
        rr.append(r); ss.append(s); oo.append(r if not p1_ else 0)
        p1.append(p1_); dz.append(dz_)
        l1.append(l1_); f2.append(f2_); l2.append(l2_); da.append(da_)

    for r in range(nr):
        for c in range(ncb):
            add(r, c * cwb, 1, 1 if cwb * (c + 1) <= bm * r else 0,
                1 if c == ncb - 1 else 0, 0, 0)
    for r in range(nr):
        cnt = (bm * r) // cwb
        if cnt >= ncb:
            add(r, (ncb - 1) * cwb, 0, 0, 0, 1, 1, 0)
        else:
            for c in range(cnt, ncb):
                add(r, c * cwb, 0, 0, 0,
                    1 if c == cnt else 0, 1 if c == ncb - 1 else 0, 1)
    nsteps = len(rr)

    arrs = [jnp.asarray(np.asarray(a, dtype=np.int32))
            for a in (rr, ss, oo, p1, dz, l1, f2, l2, da)]
    a16 = jax.lax.slice(adj_self, (0, sb), (n, n))
    body = functools.partial(_fused_body, bm=bm, cwb=cwb, sb=sb, sw=sw,
                             f=f, nsteps=nsteps)
    grid_spec = pltpu.PrefetchScalarGridSpec(
        num_scalar_prefetch=9,
        grid=(nsteps,),
        in_specs=[
            pl.BlockSpec(memory_space=pltpu.MemorySpace.HBM),
            pl.BlockSpec((n, f), lambda t, *_: (0, 0)),
            pl.BlockSpec((bm, sw), lambda t, rr, *_: (rr[t], 0)),
            pl.BlockSpec((f, e), lambda t, *_: (0, 0)),
        ],
        out_specs=pl.BlockSpec((bm, e),
                               lambda t, rr, ss, oo, *_: (oo[t], 0)),
        scratch_shapes=[
            pltpu.VMEM((4, bm, cwb), jnp.float32),
            pltpu.VMEM((n, 2 * f), jnp.float32),
            pltpu.VMEM((n, f), jnp.float32),
            pltpu.VMEM((bm, f), jnp.float32),
            pltpu.SemaphoreType.DMA((4,)),
        ],
    )
    return pl.pallas_call(
        body,
        grid_spec=grid_spec,
        out_shape=jax.ShapeDtypeStruct((n, e), jnp.float32),
    )(*arrs, adj_self, feature, a16, weight)


def _pick_bm(n):
    for bm in (512, 400, 256, 200, 128, 80, 64, 40, 16, 8):
        if n % bm == 0:
            return bm
    return n


def _spmm_kernel(a_ref, x_ref, o_ref):
    o_ref[...] = jnp.dot(a_ref[...], x_ref[...],
                         preferred_element_type=jnp.float32)


def _spmm_proj_kernel(a_ref, y_ref, w_ref, o_ref):
    t = jnp.dot(a_ref[...], y_ref[...], preferred_element_type=jnp.float32)
    o_ref[...] = 0.25 * jnp.dot(t, w_ref[...],
                                preferred_element_type=jnp.float32)


def _two_pass(feature, adj_self, weight):
    n, f = feature.shape
    e = weight.shape[1]
    bm = _pick_bm(n)
    grid = (n // bm,)
    params = pltpu.CompilerParams(dimension_semantics=("parallel",))
    y = pl.pallas_call(
        _spmm_kernel,
        grid=grid,
        in_specs=[
            pl.BlockSpec((bm, n), lambda i: (i, 0)),
            pl.BlockSpec((n, f), lambda i: (0, 0)),
        ],
        out_specs=pl.BlockSpec((bm, f), lambda i: (i, 0)),
        out_shape=jax.ShapeDtypeStruct((n, f), jnp.float32),
        compiler_params=params,
    )(adj_self, feature)
    z = pl.pallas_call(
        _spmm_proj_kernel,
        grid=grid,
        in_specs=[
            pl.BlockSpec((bm, n), lambda i: (i, 0)),
            pl.BlockSpec((n, f), lambda i: (0, 0)),
            pl.BlockSpec((f, e), lambda i: (0, 0)),
        ],
        out_specs=pl.BlockSpec((bm, e), lambda i: (i, 0)),
        out_shape=jax.ShapeDtypeStruct((n, e), jnp.float32),
        compiler_params=params,
    )(adj_self, y, weight)
    return z


def kernel(feature, adj_self, weight):
    n = feature.shape[0]
    if (adj_self.shape == (n, n) and n % 25 == 0 and (n // 25) % 8 == 0
            and n % 128 != 0 and (n % 128) % 8 == 0):
        t128 = (n - n % 128) // 128
        for ncb in (3, 2, 4, 6, 5, 7, 8, 9, 10, 13):
            if t128 % ncb == 0:
                cwb = (t128 // ncb) * 128
                return _fused(feature, adj_self, weight, n // 25, cwb)
    return _two_pass(feature, adj_self, weight)


# speedup vs baseline: 1.0244x; 1.0242x over previous
import functools

import numpy as np

import jax
import jax.numpy as jnp
from jax.experimental import pallas as pl
from jax.experimental.pallas import tpu as pltpu

_NBUF = 3


def _fused_body(rr, ss, oo, p1, l1, f2, l2, mk, sl,
                a_hbm, x_ref, a16_ref, w_ref, o_ref,
                abuf, bbuf, xy_sc, z_sc, stage, sem_a, sem_b,
                *, bm, cwb, cw2, sb, sw, f, nsteps):
    t = pl.program_id(0)
    r = rr[t]
    s = pl.multiple_of(ss[t], 128)
    rb = pl.multiple_of(r * bm, 8)

    def _copy(step, which):
        slot = sl[step]
        if which == 0:
            return pltpu.make_async_copy(
                a_hbm.at[pl.ds(pl.multiple_of(rr[step] * bm, 8), bm),
                         pl.ds(pl.multiple_of(ss[step], 128), cwb)],
                abuf.at[slot], sem_a.at[slot])
        return pltpu.make_async_copy(
            a_hbm.at[pl.ds(pl.multiple_of(rr[step] * bm, 8), bm),
                     pl.ds(pl.multiple_of(ss[step], 128), cw2)],
            bbuf.at[slot], sem_b.at[slot])

    def _issue(step):
        @pl.when(p1[step] == 1)
        def _():
            _copy(step, 0).start()

        @pl.when(p1[step] == 0)
        def _():
            _copy(step, 1).start()

    @pl.when(t == 0)
    def _():
        z_sc[...] = jnp.zeros(z_sc.shape, z_sc.dtype)
        xy_sc[:, f:] = jnp.zeros((xy_sc.shape[0], f), xy_sc.dtype)
        xy_sc[:, :f] = x_ref[...]
        _issue(0)
        _issue(1)

    @pl.when(t + 2 < nsteps)
    def _():
        _issue(t + 2)

    def _work1(ab_ref):
        term = jnp.dot(ab_ref[...], xy_sc[pl.ds(s, cwb), :],
                       preferred_element_type=jnp.float32)

        @pl.when(s == 0)
        def _():
            stage[...] = term[:, :f]

        @pl.when(s != 0)
        def _():
            stage[...] += term[:, :f]

        z_sc[pl.ds(rb, bm), :] += term[:, f:]

    def _work2(bb_ref):
        @pl.when(mk[t] == 1)
        def _():
            cut = r * bm - ss[t]
            ii = jax.lax.broadcasted_iota(jnp.int32, (cw2, f), 0)
            yv = jnp.where(ii < cut, 0.0, xy_sc[pl.ds(s, cw2), f:])
            o_ref[...] += jnp.dot(bb_ref[...], yv,
                                  preferred_element_type=jnp.float32)

        @pl.when(mk[t] == 0)
        def _():
            o_ref[...] += jnp.dot(bb_ref[...], xy_sc[pl.ds(s, cw2), f:],
                                  preferred_element_type=jnp.float32)

    @pl.when(f2[t] == 1)
    def _():
        o_ref[...] = z_sc[pl.ds(rb, bm), :]

    @pl.when(p1[t] == 1)
    def _():
        _copy(t, 0).wait()
        for k in range(_NBUF):
            @pl.when(sl[t] == k)
            def _(k=k):
                _work1(abuf.at[k])

    @pl.when(p1[t] == 0)
    def _():
        _copy(t, 1).wait()
        for k in range(_NBUF):
            @pl.when(sl[t] == k)
            def _(k=k):
                _work2(bbuf.at[k])

    @pl.when(l1[t] == 1)
    def _():
        stage[...] += jnp.dot(a16_ref[...], x_ref[pl.ds(sb, sw), :],
                              preferred_element_type=jnp.float32)
        xy_sc[pl.ds(rb, bm), f:] = stage[...]

    @pl.when(l2[t] == 1)
    def _():
        acc = o_ref[...] + jnp.dot(a16_ref[...], xy_sc[pl.ds(sb, sw), f:],
                                   preferred_element_type=jnp.float32)
        o_ref[...] = 0.25 * jnp.dot(acc, w_ref[...],
                                    preferred_element_type=jnp.float32)


def _fused(feature, adj_self, weight, bm, cwb, cw2):
    n, f = feature.shape
    e = weight.shape[1]
    nr = n // bm
    sb = (n // cwb) * cwb if n % cwb else n - cwb
    ncb = sb // cwb
    nc2 = sb // cw2
    sw = n - sb

    rr, ss, oo, p1, l1, f2, l2, mk, sl = ([] for _ in range(9))
    cnt_a = cnt_b = 0

    def add(r, s, p1_, l1_, f2_, l2_, mk_):
        nonlocal cnt_a, cnt_b
        rr.append(r); ss.append(s); oo.append(r if not p1_ else 0)
        p1.append(p1_); l1.append(l1_); f2.append(f2_); l2.append(l2_)
        mk.append(mk_)
        if p1_:
            sl.append(cnt_a % _NBUF); cnt_a += 1
        else:
            sl.append(cnt_b % _NBUF); cnt_b += 1

    for r in range(nr):
        for c in range(ncb):
            add(r, c * cwb, 1, 1 if c == ncb - 1 else 0, 0, 0, 0)
    for r in range(nr):
        c0 = min(nc2 - 1, (bm * r) // cw2)
        for c in range(c0, nc2):
            add(r, c * cw2, 0, 0,
                1 if c == c0 else 0, 1 if c == nc2 - 1 else 0,
                1 if (c == c0 and bm * r - c0 * cw2 > 0) else 0)
    nsteps = len(rr)

    arrs = [jnp.asarray(np.asarray(a, dtype=np.int32))
            for a in (rr, ss, oo, p1, l1, f2, l2, mk, sl)]
    a16 = jax.lax.slice(adj_self, (0, sb), (n, n))
    body = functools.partial(_fused_body, bm=bm, cwb=cwb, cw2=cw2,
                             sb=sb, sw=sw, f=f, nsteps=nsteps)
    grid_spec = pltpu.PrefetchScalarGridSpec(
        num_scalar_prefetch=9,
        grid=(nsteps,),
        in_specs=[
            pl.BlockSpec(memory_space=pltpu.MemorySpace.HBM),
            pl.BlockSpec((n, f), lambda t, *_: (0, 0)),
            pl.BlockSpec((bm, sw), lambda t, rr, *_: (rr[t], 0)),
            pl.BlockSpec((f, e), lambda t, *_: (0, 0)),
        ],
        out_specs=pl.BlockSpec((bm, e),
                               lambda t, rr, ss, oo, *_: (oo[t], 0)),
        scratch_shapes=[
            pltpu.VMEM((_NBUF, bm, cwb), jnp.float32),
            pltpu.VMEM((_NBUF, bm, cw2), jnp.float32),
            pltpu.VMEM((n, 2 * f), jnp.float32),
            pltpu.VMEM((n, f), jnp.float32),
            pltpu.VMEM((bm, f), jnp.float32),
            pltpu.SemaphoreType.DMA((_NBUF,)),
            pltpu.SemaphoreType.DMA((_NBUF,)),
        ],
    )
    return pl.pallas_call(
        body,
        grid_spec=grid_spec,
        out_shape=jax.ShapeDtypeStruct((n, e), jnp.float32),
    )(*arrs, adj_self, feature, a16, weight)


def _pick_bm(n):
    for bm in (512, 400, 256, 200, 128, 80, 64, 40, 16, 8):
        if n % bm == 0:
            return bm
    return n


def _spmm_kernel(a_ref, x_ref, o_ref):
    o_ref[...] = jnp.dot(a_ref[...], x_ref[...],
                         preferred_element_type=jnp.float32)


def _spmm_proj_kernel(a_ref, y_ref, w_ref, o_ref):
    t = jnp.dot(a_ref[...], y_ref[...], preferred_element_type=jnp.float32)
    o_ref[...] = 0.25 * jnp.dot(t, w_ref[...],
                                preferred_element_type=jnp.float32)


def _two_pass(feature, adj_self, weight):
    n, f = feature.shape
    e = weight.shape[1]
    bm = _pick_bm(n)
    grid = (n // bm,)
    params = pltpu.CompilerParams(dimension_semantics=("parallel",))
    y = pl.pallas_call(
        _spmm_kernel,
        grid=grid,
        in_specs=[
            pl.BlockSpec((bm, n), lambda i: (i, 0)),
            pl.BlockSpec((n, f), lambda i: (0, 0)),
        ],
        out_specs=pl.BlockSpec((bm, f), lambda i: (i, 0)),
        out_shape=jax.ShapeDtypeStruct((n, f), jnp.float32),
        compiler_params=params,
    )(adj_self, feature)
    z = pl.pallas_call(
        _spmm_proj_kernel,
        grid=grid,
        in_specs=[
            pl.BlockSpec((bm, n), lambda i: (i, 0)),
            pl.BlockSpec((n, f), lambda i: (0, 0)),
            pl.BlockSpec((f, e), lambda i: (0, 0)),
        ],
        out_specs=pl.BlockSpec((bm, e), lambda i: (i, 0)),
        out_shape=jax.ShapeDtypeStruct((n, e), jnp.float32),
        compiler_params=params,
    )(adj_self, y, weight)
    return z


def _divisors(x):
    return [d for d in range(1, x + 1) if x % d == 0]


def kernel(feature, adj_self, weight):
    n = feature.shape[0]
    if (adj_self.shape == (n, n) and n % 25 == 0 and (n // 25) % 8 == 0
            and n % 128 != 0 and (n % 128) % 8 == 0):
        t128 = (n - n % 128) // 128
        divs = _divisors(t128)
        ncb = min((d for d in divs if d >= 2), default=None,
                  key=lambda d: abs(d - 3))
        nc2 = min((d for d in divs if d >= 2), default=None,
                  key=lambda d: abs(d - 6))
        if ncb is not None and nc2 is not None and nc2 >= ncb:
            cwb = (t128 // ncb) * 128
            cw2 = (t128 // nc2) * 128
            return _fused(feature, adj_self, weight, n // 25, cwb, cw2)
    return _two_pass(feature, adj_self, weight)


# speedup vs baseline: 1.0386x; 1.0139x over previous
import functools

import numpy as np

import jax
import jax.numpy as jnp
from jax.experimental import pallas as pl
from jax.experimental.pallas import tpu as pltpu

_NBUF = 3


def _fused_body(rr, ss, oo, p1, l1, f2, l2, mk, sl,
                a_hbm, x_ref, a16_ref, w_ref, o_ref,
                abuf, bbuf, xy_sc, z_sc, stage, sem_a, sem_b,
                *, bm, cwb, cw2, sb, sw, f, nsteps):
    t = pl.program_id(0)
    r = rr[t]
    s = pl.multiple_of(ss[t], 128)
    rb = pl.multiple_of(r * bm, 8)

    def _copy(step, which):
        slot = sl[step]
        if which == 0:
            return pltpu.make_async_copy(
                a_hbm.at[pl.ds(pl.multiple_of(rr[step] * bm, 8), bm),
                         pl.ds(pl.multiple_of(ss[step], 128), cwb)],
                abuf.at[slot], sem_a.at[slot])
        return pltpu.make_async_copy(
            a_hbm.at[pl.ds(pl.multiple_of(rr[step] * bm, 8), bm),
                     pl.ds(pl.multiple_of(ss[step], 128), cw2)],
            bbuf.at[slot], sem_b.at[slot])

    def _issue(step):
        @pl.when(p1[step] == 1)
        def _():
            _copy(step, 0).start()

        @pl.when(p1[step] == 0)
        def _():
            _copy(step, 1).start()

    @pl.when(t == 0)
    def _():
        z_sc[...] = jnp.zeros(z_sc.shape, z_sc.dtype)
        xy_sc[:, f:] = jnp.zeros((xy_sc.shape[0], f), xy_sc.dtype)
        xy_sc[:, :f] = x_ref[...]
        _issue(0)
        _issue(1)

    @pl.when(t + 2 < nsteps)
    def _():
        _issue(t + 2)

    def _work1(ab_ref):
        term = jnp.dot(ab_ref[...], xy_sc[pl.ds(s, cwb), :],
                       preferred_element_type=jnp.float32)

        @pl.when(s == 0)
        def _():
            stage[...] = term[:, :f]

        @pl.when(s != 0)
        def _():
            stage[...] += term[:, :f]

        z_sc[pl.ds(rb, bm), :] += term[:, f:]

    def _work2(bb_ref):
        @pl.when(mk[t] == 1)
        def _():
            cut = r * bm - ss[t]
            ii = jax.lax.broadcasted_iota(jnp.int32, (cw2, f), 0)
            yv = jnp.where(ii < cut, 0.0, xy_sc[pl.ds(s, cw2), f:])
            o_ref[...] += jnp.dot(bb_ref[...], yv,
                                  preferred_element_type=jnp.float32)

        @pl.when(mk[t] == 0)
        def _():
            o_ref[...] += jnp.dot(bb_ref[...], xy_sc[pl.ds(s, cw2), f:],
                                  preferred_element_type=jnp.float32)

    @pl.when(f2[t] == 1)
    def _():
        o_ref[...] = z_sc[pl.ds(rb, bm), :]

    @pl.when(p1[t] == 1)
    def _():
        _copy(t, 0).wait()
        for k in range(_NBUF):
            @pl.when(sl[t] == k)
            def _(k=k):
                _work1(abuf.at[k])

    @pl.when(p1[t] == 0)
    def _():
        _copy(t, 1).wait()
        for k in range(_NBUF):
            @pl.when(sl[t] == k)
            def _(k=k):
                _work2(bbuf.at[k])

    @pl.when(l1[t] == 1)
    def _():
        stage[...] += jnp.dot(a16_ref[...], x_ref[pl.ds(sb, sw), :],
                              preferred_element_type=jnp.float32)
        xy_sc[pl.ds(rb, bm), f:] = stage[...]

    @pl.when(l2[t] == 1)
    def _():
        acc = o_ref[...] + jnp.dot(a16_ref[...], xy_sc[pl.ds(sb, sw), f:],
                                   preferred_element_type=jnp.float32)
        o_ref[...] = 0.25 * jnp.dot(acc, w_ref[...],
                                    preferred_element_type=jnp.float32)


def _fused(feature, adj_self, weight, bm, cwb, cw2):
    n, f = feature.shape
    e = weight.shape[1]
    nr = n // bm
    sb = (n // cwb) * cwb if n % cwb else n - cwb
    ncb = sb // cwb
    nc2 = sb // cw2
    sw = n - sb

    rr, ss, oo, p1, l1, f2, l2, mk, sl = ([] for _ in range(9))
    cnt_a = cnt_b = 0

    def add(r, s, p1_, l1_, f2_, l2_, mk_):
        nonlocal cnt_a, cnt_b
        rr.append(r); ss.append(s); oo.append(r if not p1_ else 0)
        p1.append(p1_); l1.append(l1_); f2.append(f2_); l2.append(l2_)
        mk.append(mk_)
        if p1_:
            sl.append(cnt_a % _NBUF); cnt_a += 1
        else:
            sl.append(cnt_b % _NBUF); cnt_b += 1

    for r in range(nr):
        for c in range(ncb):
            add(r, c * cwb, 1, 1 if c == ncb - 1 else 0, 0, 0, 0)
    for r in range(nr):
        c0 = min(nc2 - 1, (bm * r) // cw2)
        for c in range(c0, nc2):
            add(r, c * cw2, 0, 0,
                1 if c == c0 else 0, 1 if c == nc2 - 1 else 0,
                1 if (c == c0 and bm * r - c0 * cw2 > 0) else 0)
    nsteps = len(rr)

    arrs = [jnp.asarray(np.asarray(a, dtype=np.int32))
            for a in (rr, ss, oo, p1, l1, f2, l2, mk, sl)]
    a16 = jax.lax.slice(adj_self, (0, sb), (n, n))
    body = functools.partial(_fused_body, bm=bm, cwb=cwb, cw2=cw2,
                             sb=sb, sw=sw, f=f, nsteps=nsteps)
    grid_spec = pltpu.PrefetchScalarGridSpec(
        num_scalar_prefetch=9,
        grid=(nsteps,),
        in_specs=[
            pl.BlockSpec(memory_space=pltpu.MemorySpace.HBM),
            pl.BlockSpec((n, f), lambda t, *_: (0, 0)),
            pl.BlockSpec((bm, sw), lambda t, rr, *_: (rr[t], 0)),
            pl.BlockSpec((f, e), lambda t, *_: (0, 0)),
        ],
        out_specs=pl.BlockSpec((bm, e),
                               lambda t, rr, ss, oo, *_: (oo[t], 0)),
        scratch_shapes=[
            pltpu.VMEM((_NBUF, bm, cwb), jnp.float32),
            pltpu.VMEM((_NBUF, bm, cw2), jnp.float32),
            pltpu.VMEM((n, 2 * f), jnp.float32),
            pltpu.VMEM((n, f), jnp.float32),
            pltpu.VMEM((bm, f), jnp.float32),
            pltpu.SemaphoreType.DMA((_NBUF,)),
            pltpu.SemaphoreType.DMA((_NBUF,)),
        ],
    )
    return pl.pallas_call(
        body,
        grid_spec=grid_spec,
        out_shape=jax.ShapeDtypeStruct((n, e), jnp.float32),
    )(*arrs, adj_self, feature, a16, weight)


def _pick_bm(n):
    for bm in (512, 400, 256, 200, 128, 80, 64, 40, 16, 8):
        if n % bm == 0:
            return bm
    return n


def _spmm_kernel(a_ref, x_ref, o_ref):
    o_ref[...] = jnp.dot(a_ref[...], x_ref[...],
                         preferred_element_type=jnp.float32)


def _spmm_proj_kernel(a_ref, y_ref, w_ref, o_ref):
    t = jnp.dot(a_ref[...], y_ref[...], preferred_element_type=jnp.float32)
    o_ref[...] = 0.25 * jnp.dot(t, w_ref[...],
                                preferred_element_type=jnp.float32)


def _two_pass(feature, adj_self, weight):
    n, f = feature.shape
    e = weight.shape[1]
    bm = _pick_bm(n)
    grid = (n // bm,)
    params = pltpu.CompilerParams(dimension_semantics=("parallel",))
    y = pl.pallas_call(
        _spmm_kernel,
        grid=grid,
        in_specs=[
            pl.BlockSpec((bm, n), lambda i: (i, 0)),
            pl.BlockSpec((n, f), lambda i: (0, 0)),
        ],
        out_specs=pl.BlockSpec((bm, f), lambda i: (i, 0)),
        out_shape=jax.ShapeDtypeStruct((n, f), jnp.float32),
        compiler_params=params,
    )(adj_self, feature)
    z = pl.pallas_call(
        _spmm_proj_kernel,
        grid=grid,
        in_specs=[
            pl.BlockSpec((bm, n), lambda i: (i, 0)),
            pl.BlockSpec((n, f), lambda i: (0, 0)),
            pl.BlockSpec((f, e), lambda i: (0, 0)),
        ],
        out_specs=pl.BlockSpec((bm, e), lambda i: (i, 0)),
        out_shape=jax.ShapeDtypeStruct((n, e), jnp.float32),
        compiler_params=params,
    )(adj_self, y, weight)
    return z


def _divisors(x):
    return [d for d in range(1, x + 1) if x % d == 0]


def kernel(feature, adj_self, weight):
    n = feature.shape[0]
    if (adj_self.shape == (n, n) and n % 25 == 0 and (n // 25) % 8 == 0
            and n % 128 != 0 and (n % 128) % 8 == 0):
        t128 = (n - n % 128) // 128
        divs = _divisors(t128)
        ncb = min((d for d in divs if d >= 2), default=None,
                  key=lambda d: abs(d - 3))
        nc2 = min((d for d in divs if d >= 2), default=None,
                  key=lambda d: abs(d - 3))
        if ncb is not None and nc2 is not None and nc2 >= ncb:
            cwb = (t128 // ncb) * 128
            cw2 = (t128 // nc2) * 128
            return _fused(feature, adj_self, weight, n // 25, cwb, cw2)
    return _two_pass(feature, adj_self, weight)


# speedup vs baseline: 1.0816x; 1.0414x over previous
import functools

import numpy as np

import jax
import jax.numpy as jnp
from jax.experimental import pallas as pl
from jax.experimental.pallas import tpu as pltpu

_NBUF = 3


def _fused_body(rr, ss, oo, p1, l1, f2, l2, mk, sl, wd,
                a_hbm, x_ref, a16_ref, w_ref, o_ref,
                abuf, bbuf, xy_sc, z_sc, stage, sem_a, sem_b,
                *, bm, cwb, cw2, sb, sw, f, nsteps):
    t = pl.program_id(0)
    r = rr[t]
    s = pl.multiple_of(ss[t], 128)
    rb = pl.multiple_of(r * bm, 8)

    def _copy(step, which):
        slot = sl[step]
        if which == 0:
            return pltpu.make_async_copy(
                a_hbm.at[pl.ds(pl.multiple_of(rr[step] * bm, 8), bm),
                         pl.ds(pl.multiple_of(ss[step], 128), cwb)],
                abuf.at[slot], sem_a.at[slot])
        return pltpu.make_async_copy(
            a_hbm.at[pl.ds(pl.multiple_of(rr[step] * bm, 8), bm),
                     pl.ds(pl.multiple_of(ss[step], 128), cw2)],
            bbuf.at[slot], sem_b.at[slot])

    def _issue(step):
        @pl.when(wd[step] == 0)
        def _():
            _copy(step, 0).start()

        @pl.when(wd[step] == 1)
        def _():
            _copy(step, 1).start()

    @pl.when(t == 0)
    def _():
        z_sc[...] = jnp.zeros(z_sc.shape, z_sc.dtype)
        xy_sc[:, f:] = jnp.zeros((xy_sc.shape[0], f), xy_sc.dtype)
        xy_sc[:, :f] = x_ref[...]
        _issue(0)
        _issue(1)

    @pl.when(t + 2 < nsteps)
    def _():
        _issue(t + 2)

    def _work1(ab_ref):
        term = jnp.dot(ab_ref[...], xy_sc[pl.ds(s, cwb), :],
                       preferred_element_type=jnp.float32)

        @pl.when(s == 0)
        def _():
            stage[...] = term[:, :f]

        @pl.when(s != 0)
        def _():
            stage[...] += term[:, :f]

        z_sc[pl.ds(rb, bm), :] += term[:, f:]

    def _work2(bb_ref, cw):
        @pl.when(mk[t] == 1)
        def _():
            cut = r * bm - ss[t]
            ii = jax.lax.broadcasted_iota(jnp.int32, (cw, f), 0)
            yv = jnp.where(ii < cut, 0.0, xy_sc[pl.ds(s, cw), f:])
            o_ref[...] += jnp.dot(bb_ref[...], yv,
                                  preferred_element_type=jnp.float32)

        @pl.when(mk[t] == 0)
        def _():
            o_ref[...] += jnp.dot(bb_ref[...], xy_sc[pl.ds(s, cw), f:],
                                  preferred_element_type=jnp.float32)

    @pl.when(f2[t] == 1)
    def _():
        o_ref[...] = z_sc[pl.ds(rb, bm), :]

    @pl.when(p1[t] == 1)
    def _():
        _copy(t, 0).wait()
        for k in range(_NBUF):
            @pl.when(sl[t] == k)
            def _(k=k):
                _work1(abuf.at[k])

    @pl.when(jnp.logical_and(p1[t] == 0, wd[t] == 0))
    def _():
        _copy(t, 0).wait()
        for k in range(_NBUF):
            @pl.when(sl[t] == k)
            def _(k=k):
                _work2(abuf.at[k], cwb)

    @pl.when(wd[t] == 1)
    def _():
        _copy(t, 1).wait()
        for k in range(_NBUF):
            @pl.when(sl[t] == k)
            def _(k=k):
                _work2(bbuf.at[k], cw2)

    @pl.when(l1[t] == 1)
    def _():
        stage[...] += jnp.dot(a16_ref[...], x_ref[pl.ds(sb, sw), :],
                              preferred_element_type=jnp.float32)
        xy_sc[pl.ds(rb, bm), f:] = stage[...]

    @pl.when(l2[t] == 1)
    def _():
        acc = o_ref[...] + jnp.dot(a16_ref[...], xy_sc[pl.ds(sb, sw), f:],
                                   preferred_element_type=jnp.float32)
        o_ref[...] = 0.25 * jnp.dot(acc, w_ref[...],
                                    preferred_element_type=jnp.float32)


def _fused(feature, adj_self, weight, bm, cwb, cw2):
    n, f = feature.shape
    e = weight.shape[1]
    nr = n // bm
    sb = (n // cwb) * cwb if n % cwb else n - cwb
    ncb = sb // cwb
    nc2 = sb // cw2
    sw = n - sb

    rr, ss, oo, p1, l1, f2, l2, mk, sl, wd = ([] for _ in range(10))
    cnt_a = cnt_b = 0
    halved = cw2 * 2 == cwb

    def add(r, s, p1_, l1_, f2_, l2_, mk_, wd_=0):
        nonlocal cnt_a, cnt_b
        rr.append(r); ss.append(s); oo.append(r if not p1_ else 0)
        p1.append(p1_); l1.append(l1_); f2.append(f2_); l2.append(l2_)
        mk.append(mk_); wd.append(wd_)
        if wd_ == 0:
            sl.append(cnt_a % _NBUF); cnt_a += 1
        else:
            sl.append(cnt_b % _NBUF); cnt_b += 1

    for r in range(nr):
        for c in range(ncb):
            add(r, c * cwb, 1, 1 if c == ncb - 1 else 0, 0, 0, 0)
    for r in range(nr):
        b = bm * r
        c0 = min(ncb - 1, b // cwb)
        rem = b - c0 * cwb
        steps = []
        if halved and rem >= cw2:
            steps.append((c0 * cwb + cw2, 1, rem - cw2))
        else:
            steps.append((c0 * cwb, 0, rem))
        for c in range(c0 + 1, ncb):
            steps.append((c * cwb, 0, 0))
        for i, (s_, wd_, cut) in enumerate(steps):
            add(r, s_, 0, 0,
                1 if i == 0 else 0, 1 if i == len(steps) - 1 else 0,
                1 if (i == 0 and cut > 0) else 0, wd_)
    nsteps = len(rr)

    arrs = [jnp.asarray(np.asarray(a, dtype=np.int32))
            for a in (rr, ss, oo, p1, l1, f2, l2, mk, sl, wd)]
    a16 = jax.lax.slice(adj_self, (0, sb), (n, n))
    body = functools.partial(_fused_body, bm=bm, cwb=cwb, cw2=cw2,
                             sb=sb, sw=sw, f=f, nsteps=nsteps)
    grid_spec = pltpu.PrefetchScalarGridSpec(
        num_scalar_prefetch=10,
        grid=(nsteps,),
        in_specs=[
            pl.BlockSpec(memory_space=pltpu.MemorySpace.HBM),
            pl.BlockSpec((n, f), lambda t, *_: (0, 0)),
            pl.BlockSpec((bm, sw), lambda t, rr, *_: (rr[t], 0)),
            pl.BlockSpec((f, e), lambda t, *_: (0, 0)),
        ],
        out_specs=pl.BlockSpec((bm, e),
                               lambda t, rr, ss, oo, *_: (oo[t], 0)),
        scratch_shapes=[
            pltpu.VMEM((_NBUF, bm, cwb), jnp.float32),
            pltpu.VMEM((_NBUF, bm, cw2), jnp.float32),
            pltpu.VMEM((n, 2 * f), jnp.float32),
            pltpu.VMEM((n, f), jnp.float32),
            pltpu.VMEM((bm, f), jnp.float32),
            pltpu.SemaphoreType.DMA((_NBUF,)),
            pltpu.SemaphoreType.DMA((_NBUF,)),
        ],
    )
    return pl.pallas_call(
        body,
        grid_spec=grid_spec,
        out_shape=jax.ShapeDtypeStruct((n, e), jnp.float32),
    )(*arrs, adj_self, feature, a16, weight)


def _pick_bm(n):
    for bm in (512, 400, 256, 200, 128, 80, 64, 40, 16, 8):
        if n % bm == 0:
            return bm
    return n


def _spmm_kernel(a_ref, x_ref, o_ref):
    o_ref[...] = jnp.dot(a_ref[...], x_ref[...],
                         preferred_element_type=jnp.float32)


def _spmm_proj_kernel(a_ref, y_ref, w_ref, o_ref):
    t = jnp.dot(a_ref[...], y_ref[...], preferred_element_type=jnp.float32)
    o_ref[...] = 0.25 * jnp.dot(t, w_ref[...],
                                preferred_element_type=jnp.float32)


def _two_pass(feature, adj_self, weight):
    n, f = feature.shape
    e = weight.shape[1]
    bm = _pick_bm(n)
    grid = (n // bm,)
    params = pltpu.CompilerParams(dimension_semantics=("parallel",))
    y = pl.pallas_call(
        _spmm_kernel,
        grid=grid,
        in_specs=[
            pl.BlockSpec((bm, n), lambda i: (i, 0)),
            pl.BlockSpec((n, f), lambda i: (0, 0)),
        ],
        out_specs=pl.BlockSpec((bm, f), lambda i: (i, 0)),
        out_shape=jax.ShapeDtypeStruct((n, f), jnp.float32),
        compiler_params=params,
    )(adj_self, feature)
    z = pl.pallas_call(
        _spmm_proj_kernel,
        grid=grid,
        in_specs=[
            pl.BlockSpec((bm, n), lambda i: (i, 0)),
            pl.BlockSpec((n, f), lambda i: (0, 0)),
            pl.BlockSpec((f, e), lambda i: (0, 0)),
        ],
        out_specs=pl.BlockSpec((bm, e), lambda i: (i, 0)),
        out_shape=jax.ShapeDtypeStruct((n, e), jnp.float32),
        compiler_params=params,
    )(adj_self, y, weight)
    return z


def _divisors(x):
    return [d for d in range(1, x + 1) if x % d == 0]


def kernel(feature, adj_self, weight):
    n = feature.shape[0]
    if (adj_self.shape == (n, n) and n % 25 == 0 and (n // 25) % 8 == 0
            and n % 128 != 0 and (n % 128) % 8 == 0):
        t128 = (n - n % 128) // 128
        divs = _divisors(t128)
        ncb = min((d for d in divs if d >= 2), default=None,
                  key=lambda d: abs(d - 3))
        if ncb is not None:
            cwb = (t128 // ncb) * 128
            cw2 = cwb // 2 if (cwb // 128) % 2 == 0 else cwb
            return _fused(feature, adj_self, weight, n // 25, cwb, cw2)
    return _two_pass(feature, adj_self, weight)
